# baseline (speedup 1.0000x reference)
import functools

import jax
import jax.numpy as jnp
from jax import lax
from jax.experimental import pallas as pl
from jax.experimental.pallas import tpu as pltpu
from jax.experimental.pallas import tpu_sc as plsc

_N = 100000
_W = 256
_R = 80
_NCHUNK = _N // _R
_NC = 2
_NS = 16
_NW = _NC * _NS
_LANES = 16
_JSTEPS = _W // _LANES


def _sc_body(x_hbm, deg_hbm, scale_hbm, out_hbm, x_v, o_v, deg_v,
             sraw_v, es_v):
    wid = lax.axis_index("s") * _NC + lax.axis_index("c")

    pltpu.sync_copy(scale_hbm, sraw_v)
    for k in range(4 * _W // _LANES):
        row, col = divmod(k * _LANES, _W)
        es_v[pl.ds(k * _LANES, _LANES)] = jnp.exp(
            sraw_v[row, pl.ds(col, _LANES)])

    iota = lax.broadcasted_iota(jnp.int32, (_LANES,), 0)

    def do_chunk(c):
        base = c * _R
        pltpu.sync_copy(x_hbm.at[pl.ds(base, _R)], x_v)
        pltpu.sync_copy(deg_hbm.at[pl.ds(base, _R)], deg_v)

        def do_group(g, carry):
            dv = deg_v[pl.ds(g * _LANES, _LANES)]
            for rr in range(_LANES):
                r = g * _LANES + rr
                mbase = dv[rr] * _W
                for j in range(_JSTEPS):
                    m = es_v[pl.ds(mbase + j * _LANES, _LANES)]
                    o_v[r, pl.ds(j * _LANES, _LANES)] = (
                        x_v[r, pl.ds(j * _LANES, _LANES)] * m)
            return carry

        lax.fori_loop(0, _R // _LANES, do_group, 0)
        pltpu.sync_copy(o_v, out_hbm.at[pl.ds(base, _R)])

    nfull = _NCHUNK // _NW
    nextra = _NCHUNK - nfull * _NW
    count = nfull + jnp.where(wid < nextra, 1, 0)

    def loop(i, carry):
        do_chunk(wid + i * _NW)
        return carry

    lax.fori_loop(0, count, loop, 0)


def kernel(x, deg, scale):
    n, w = x.shape
    mesh = plsc.VectorSubcoreMesh(core_axis_name="c", subcore_axis_name="s")
    run = pl.kernel(
        _sc_body,
        out_type=jax.ShapeDtypeStruct((n, w), x.dtype),
        mesh=mesh,
        scratch_types=[
            pltpu.VMEM((_R, _W), jnp.float32),
            pltpu.VMEM((_R, _W), jnp.float32),
            pltpu.VMEM((_R,), jnp.int32),
            pltpu.VMEM((4, _W), jnp.float32),
            pltpu.VMEM((4 * _W,), jnp.float32),
        ],
    )
    return run(x, deg.astype(jnp.int32), scale)


# speedup vs baseline: 1.3782x; 1.3782x over previous
import functools

import jax
import jax.numpy as jnp
from jax import lax
from jax.experimental import pallas as pl
from jax.experimental.pallas import tpu as pltpu
from jax.experimental.pallas import tpu_sc as plsc

_N = 100000
_W = 256
_R = 80
_NCHUNK = _N // _R
_NC = 2
_NS = 16
_NW = _NC * _NS
_LANES = 16
_JSTEPS = _W // _LANES
_GROUPS = _R // _LANES
_MAXK = 40


def _sc_body(x_hbm, deg_hbm, scale_hbm, out_hbm, x_v, o_v, dg_v,
             sraw_v, es_v, sem_x, sem_d, sem_o):
    wid = lax.axis_index("s") * _NC + lax.axis_index("c")
    nfull = _NCHUNK // _NW
    nextra = _NCHUNK - nfull * _NW
    count = nfull + jnp.where(wid < nextra, 1, 0)

    pltpu.sync_copy(scale_hbm, sraw_v)
    for k in range(4 * _W // _LANES):
        row, col = divmod(k * _LANES, _W)
        es_v[pl.ds(k * _LANES, _LANES)] = jnp.exp(
            sraw_v[row, pl.ds(col, _LANES)])

    def row_base(k):
        return (wid + k * _NW) * _R

    def in_copies(k, b):
        base = row_base(k)
        return (
            pltpu.make_async_copy(
                x_hbm.at[pl.ds(base, _R)], x_v.at[b], sem_x.at[b]),
            pltpu.make_async_copy(
                deg_hbm.at[wid + k * _NW], dg_v.at[b], sem_d.at[b]),
        )

    def out_copy(k, b):
        return pltpu.make_async_copy(
            o_v.at[b], out_hbm.at[pl.ds(row_base(k), _R)], sem_o.at[b])

    def start_in(k, b):
        cx, cd = in_copies(k, b)
        cx.start()
        cd.start()

    def compute(b):
        def do_group(g, carry):
            dv = dg_v[b, pl.ds(g * _LANES, _LANES)]
            for rr in range(_LANES):
                r = g * _LANES + rr
                mbase = dv[rr] * _W
                for j in range(_JSTEPS):
                    m = es_v[pl.ds(mbase + j * _LANES, _LANES)]
                    o_v[b, r, pl.ds(j * _LANES, _LANES)] = (
                        x_v[b, r, pl.ds(j * _LANES, _LANES)] * m)
            return carry

        lax.fori_loop(0, _GROUPS, do_group, 0)

    start_in(0, 0)
    start_in(1, 1)

    def slot(k, b):
        @pl.when(k < count)
        def _():
            cx, cd = in_copies(k, b)
            cx.wait()
            cd.wait()

            @pl.when(k >= 2)
            def _():
                out_copy(k - 2, b).wait()

            compute(b)
            out_copy(k, b).start()

            @pl.when(k + 2 < count)
            def _():
                start_in(k + 2, b)

    def loop(i, carry):
        slot(2 * i, 0)
        slot(2 * i + 1, 1)
        return carry

    lax.fori_loop(0, _MAXK // 2, loop, 0)

    @pl.when(count % 2 == 0)
    def _():
        out_copy(count - 2, 0).wait()
        out_copy(count - 1, 1).wait()

    @pl.when(count % 2 == 1)
    def _():
        out_copy(count - 2, 1).wait()
        out_copy(count - 1, 0).wait()


def kernel(x, deg, scale):
    n, w = x.shape
    deg2 = deg.astype(jnp.int32).reshape(_NCHUNK, _R)
    mesh = plsc.VectorSubcoreMesh(core_axis_name="c", subcore_axis_name="s")
    run = pl.kernel(
        _sc_body,
        out_type=jax.ShapeDtypeStruct((n, w), x.dtype),
        mesh=mesh,
        scratch_types=[
            pltpu.VMEM((2, _R, _W), jnp.float32),
            pltpu.VMEM((2, _R, _W), jnp.float32),
            pltpu.VMEM((2, _R), jnp.int32),
            pltpu.VMEM((4, _W), jnp.float32),
            pltpu.VMEM((4 * _W,), jnp.float32),
            pltpu.SemaphoreType.DMA((2,)),
            pltpu.SemaphoreType.DMA((2,)),
            pltpu.SemaphoreType.DMA((2,)),
        ],
    )
    return run(x, deg2, scale)


# speedup vs baseline: 1.3787x; 1.0004x over previous
import functools

import jax
import jax.numpy as jnp
from jax import lax
from jax.experimental import pallas as pl
from jax.experimental.pallas import tpu as pltpu
from jax.experimental.pallas import tpu_sc as plsc

_N = 100000
_W = 256
_R = 80
_NCHUNK = _N // _R
_NC = 2
_NS = 16
_NW = _NC * _NS
_LANES = 16
_JSTEPS = _W // _LANES
_GROUPS = _R // _LANES
_MAXK = 40


def _sc_body(x_hbm, deg_hbm, scale_hbm, out_hbm, x_v, o_v, dg_v,
             sraw_v, es_v, sem_x, sem_d, sem_o):
    wid = lax.axis_index("s") * _NC + lax.axis_index("c")
    nfull = _NCHUNK // _NW
    nextra = _NCHUNK - nfull * _NW
    count = nfull + jnp.where(wid < nextra, 1, 0)

    pltpu.sync_copy(scale_hbm, sraw_v)
    for k in range(4 * _W // _LANES):
        row, col = divmod(k * _LANES, _W)
        es_v[pl.ds(k * _LANES, _LANES)] = jnp.exp(
            sraw_v[row, pl.ds(col, _LANES)])

    def row_base(k):
        return (wid + k * _NW) * _R

    def in_copies(k, b):
        base = row_base(k)
        return (
            pltpu.make_async_copy(
                x_hbm.at[pl.ds(base, _R)], x_v.at[b], sem_x.at[b]),
            pltpu.make_async_copy(
                deg_hbm.at[wid + k * _NW], dg_v.at[b], sem_d.at[b]),
        )

    def out_copy(k, b):
        return pltpu.make_async_copy(
            o_v.at[b], out_hbm.at[pl.ds(row_base(k), _R)], sem_o.at[b])

    def start_in(k, b):
        cx, cd = in_copies(k, b)
        cx.start()
        cd.start()

    def compute(b):
        @plsc.parallel_loop(0, _GROUPS, step=1, unroll=1)
        def do_group(g):
            dv = dg_v[b, pl.ds(g * _LANES, _LANES)]
            for rr in range(_LANES):
                r = g * _LANES + rr
                mbase = dv[rr] * _W
                for j in range(_JSTEPS):
                    m = es_v[pl.ds(mbase + j * _LANES, _LANES)]
                    o_v[b, r, pl.ds(j * _LANES, _LANES)] = (
                        x_v[b, r, pl.ds(j * _LANES, _LANES)] * m)

    start_in(0, 0)
    start_in(1, 1)

    def slot(k, b):
        @pl.when(k < count)
        def _():
            cx, cd = in_copies(k, b)
            cx.wait()
            cd.wait()

            @pl.when(k >= 2)
            def _():
                out_copy(k - 2, b).wait()

            compute(b)
            out_copy(k, b).start()

            @pl.when(k + 2 < count)
            def _():
                start_in(k + 2, b)

    def loop(i, carry):
        slot(2 * i, 0)
        slot(2 * i + 1, 1)
        return carry

    lax.fori_loop(0, _MAXK // 2, loop, 0)

    @pl.when(count % 2 == 0)
    def _():
        out_copy(count - 2, 0).wait()
        out_copy(count - 1, 1).wait()

    @pl.when(count % 2 == 1)
    def _():
        out_copy(count - 2, 1).wait()
        out_copy(count - 1, 0).wait()


def kernel(x, deg, scale):
    n, w = x.shape
    deg2 = deg.astype(jnp.int32).reshape(_NCHUNK, _R)
    mesh = plsc.VectorSubcoreMesh(core_axis_name="c", subcore_axis_name="s")
    run = pl.kernel(
        _sc_body,
        out_type=jax.ShapeDtypeStruct((n, w), x.dtype),
        mesh=mesh,
        scratch_types=[
            pltpu.VMEM((2, _R, _W), jnp.float32),
            pltpu.VMEM((2, _R, _W), jnp.float32),
            pltpu.VMEM((2, _R), jnp.int32),
            pltpu.VMEM((4, _W), jnp.float32),
            pltpu.VMEM((4 * _W,), jnp.float32),
            pltpu.SemaphoreType.DMA((2,)),
            pltpu.SemaphoreType.DMA((2,)),
            pltpu.SemaphoreType.DMA((2,)),
        ],
    )
    return run(x, deg2, scale)


# speedup vs baseline: 5.4614x; 3.9612x over previous
import jax
import jax.numpy as jnp
from jax.experimental import pallas as pl

_BLOCK_ROWS = 10000


def _body(deg_ref, scale_ref, x_ref, out_ref):
    s = jnp.exp(scale_ref[...])
    d = deg_ref[0, 0, :]
    iota = jax.lax.broadcasted_iota(jnp.int32, (1, 4), 1)
    onehot = (d[:, None] == iota).astype(jnp.float32)
    m = jnp.dot(onehot, s, preferred_element_type=jnp.float32)
    out_ref[...] = m * x_ref[...]


def kernel(x, deg, scale):
    n, w = x.shape
    b = _BLOCK_ROWS
    while n % b:
        b //= 2
    nb = n // b
    deg3 = deg.astype(jnp.int32).reshape(nb, 1, b)
    return pl.pallas_call(
        _body,
        grid=(nb,),
        in_specs=[
            pl.BlockSpec((1, 1, b), lambda i: (i, 0, 0)),
            pl.BlockSpec((4, w), lambda i: (0, 0)),
            pl.BlockSpec((b, w), lambda i: (i, 0)),
        ],
        out_specs=pl.BlockSpec((b, w), lambda i: (i, 0)),
        out_shape=jax.ShapeDtypeStruct((n, w), x.dtype),
    )(deg3, scale, x)
